# baseline (speedup 1.0000x reference)
import jax
import jax.numpy as jnp
from jax.experimental import pallas as pl
from jax.experimental.pallas import tpu as pltpu

_ROWS = 4096
_COLS = 8192
_BLOCK_ROWS = 256


def _copy_body(x_ref, len_ref, x_out_ref, len_out_ref):
    x_out_ref[...] = x_ref[...]

    @pl.when(pl.program_id(0) == 0)
    def _():
        len_out_ref[...] = len_ref[...]


def kernel(x, x_len):
    x2 = x.reshape(_ROWS, _COLS)
    len2 = x_len.reshape(1, 16)
    out_x, out_len = pl.pallas_call(
        _copy_body,
        grid=(_ROWS // _BLOCK_ROWS,),
        in_specs=[
            pl.BlockSpec((_BLOCK_ROWS, _COLS), lambda i: (i, 0)),
            pl.BlockSpec((1, 16), lambda i: (0, 0)),
        ],
        out_specs=[
            pl.BlockSpec((_BLOCK_ROWS, _COLS), lambda i: (i, 0)),
            pl.BlockSpec((1, 16), lambda i: (0, 0)),
        ],
        out_shape=[
            jax.ShapeDtypeStruct((_ROWS, _COLS), x.dtype),
            jax.ShapeDtypeStruct((1, 16), x_len.dtype),
        ],
        compiler_params=pltpu.CompilerParams(
            dimension_semantics=("arbitrary",),
        ),
    )(x2, len2)
    return out_x.reshape(x.shape), out_len.reshape(x_len.shape)


# speedup vs baseline: 4.6132x; 4.6132x over previous
import jax
import jax.numpy as jnp
from jax.experimental import pallas as pl
from jax.experimental.pallas import tpu as pltpu

_ROWS = 16 * 2048
_COLS = 1024
_CHUNK = 2048
_NCHUNKS = _ROWS // _CHUNK
_NBUF = 4


def _copy_body(x_ref, len_ref, x_out_ref, len_out_ref, *scratch):
    bufs = scratch[:_NBUF]
    len_buf = scratch[_NBUF]
    rsems = scratch[_NBUF + 1:_NBUF + 1 + _NBUF]
    wsems = scratch[_NBUF + 1 + _NBUF:]

    def src(i):
        return x_ref.at[pl.ds(i * _CHUNK, _CHUNK), :]

    def dst(i):
        return x_out_ref.at[pl.ds(i * _CHUNK, _CHUNK), :]

    len_rd = pltpu.make_async_copy(len_ref, len_buf, rsems[0])

    reads = [None] * _NCHUNKS
    writes = [None] * _NCHUNKS
    for j in range(min(_NBUF - 1, _NCHUNKS)):
        reads[j] = pltpu.make_async_copy(src(j), bufs[j % _NBUF], rsems[j % _NBUF])
        reads[j].start()
    for i in range(_NCHUNKS):
        b = i % _NBUF
        reads[i].wait()
        wr = pltpu.make_async_copy(bufs[b], dst(i), wsems[b])
        wr.start()
        writes[i] = wr
        nxt = i + _NBUF - 1
        if nxt < _NCHUNKS:
            nb = nxt % _NBUF
            if nxt >= _NBUF:
                writes[nxt - _NBUF].wait()
            reads[nxt] = pltpu.make_async_copy(src(nxt), bufs[nb], rsems[nb])
            reads[nxt].start()
    len_rd.start()
    len_rd.wait()
    len_wr = pltpu.make_async_copy(len_buf, len_out_ref, wsems[0])
    for j in range(max(0, _NCHUNKS - _NBUF), _NCHUNKS):
        writes[j].wait()
    len_wr.start()
    len_wr.wait()


def kernel(x, x_len):
    x2 = x.reshape(_ROWS, _COLS)
    len2 = x_len.reshape(1, 16)
    out_x, out_len = pl.pallas_call(
        _copy_body,
        in_specs=[
            pl.BlockSpec(memory_space=pltpu.MemorySpace.HBM),
            pl.BlockSpec(memory_space=pltpu.MemorySpace.HBM),
        ],
        out_specs=[
            pl.BlockSpec(memory_space=pltpu.MemorySpace.HBM),
            pl.BlockSpec(memory_space=pltpu.MemorySpace.HBM),
        ],
        out_shape=[
            jax.ShapeDtypeStruct((_ROWS, _COLS), x.dtype),
            jax.ShapeDtypeStruct((1, 16), x_len.dtype),
        ],
        scratch_shapes=(
            [pltpu.VMEM((_CHUNK, _COLS), jnp.float32) for _ in range(_NBUF)]
            + [pltpu.VMEM((1, 16), jnp.int32)]
            + [pltpu.SemaphoreType.DMA] * (2 * _NBUF)
        ),
        compiler_params=pltpu.CompilerParams(
            vmem_limit_bytes=48 * 1024 * 1024,
        ),
    )(x2, len2)
    return out_x.reshape(x.shape), out_len.reshape(x_len.shape)


# speedup vs baseline: 4.6608x; 1.0103x over previous
import jax
import jax.numpy as jnp
from jax.experimental import pallas as pl
from jax.experimental.pallas import tpu as pltpu

_ROWS = 16 * 2048
_COLS = 1024
_BLOCK_ROWS = 2048


def _copy_body(x_ref, len_ref, x_out_ref, len_out_ref):
    x_out_ref[...] = x_ref[...]

    @pl.when(pl.program_id(0) == 0)
    def _():
        len_out_ref[...] = len_ref[...]


def kernel(x, x_len):
    x2 = x.reshape(_ROWS, _COLS)
    len2 = x_len.reshape(1, 16)
    out_x, out_len = pl.pallas_call(
        _copy_body,
        grid=(_ROWS // _BLOCK_ROWS,),
        in_specs=[
            pl.BlockSpec((_BLOCK_ROWS, _COLS), lambda i: (i, 0)),
            pl.BlockSpec((1, 16), lambda i: (0, 0)),
        ],
        out_specs=[
            pl.BlockSpec((_BLOCK_ROWS, _COLS), lambda i: (i, 0)),
            pl.BlockSpec((1, 16), lambda i: (0, 0)),
        ],
        out_shape=[
            jax.ShapeDtypeStruct((_ROWS, _COLS), x.dtype),
            jax.ShapeDtypeStruct((1, 16), x_len.dtype),
        ],
        compiler_params=pltpu.CompilerParams(
            dimension_semantics=("arbitrary",),
        ),
    )(x2, len2)
    return out_x.reshape(x.shape), out_len.reshape(x_len.shape)
